# Initial kernel scaffold; baseline (speedup 1.0000x reference)
import jax
import jax.numpy as jnp
from jax.experimental import pallas as pl


def kernel(token_ids, embedding_mat):
    raise NotImplementedError("write your pallas kernel here")


# speedup vs baseline: 1.4584x; 1.4584x over previous
import functools

import jax
import jax.numpy as jnp
from jax import lax
from jax.experimental import pallas as pl
from jax.experimental.pallas import tpu as pltpu
from jax.experimental.pallas import tpu_sc as plsc

NUM_TOKENS = 4096 * 200
DIM = 32
NC = 2
NS = 16
NW = NC * NS
G = 128
K = 8
CHUNK = K * G
B_PER_W = NUM_TOKENS // NW
N_OUTER = B_PER_W // CHUNK
ROWS_PER_W = B_PER_W // G

_mesh = plsc.VectorSubcoreMesh(core_axis_name="c", subcore_axis_name="s")


@functools.partial(
    pl.kernel,
    out_type=jax.ShapeDtypeStruct((NUM_TOKENS, DIM), jnp.float32),
    mesh=_mesh,
    compiler_params=pltpu.CompilerParams(use_tc_tiling_on_sc=False),
    scratch_types=[
        pltpu.VMEM((K, G), jnp.int32),
        pltpu.VMEM((CHUNK, DIM), jnp.float32),
        pltpu.SemaphoreType.DMA,
    ],
)
def _gather_kernel(idx_hbm, table_hbm, out_hbm, idx_v, rows_v, sem):
    wid = lax.axis_index("s") * NC + lax.axis_index("c")
    row_base = wid * ROWS_PER_W
    out_base = wid * B_PER_W

    @pl.loop(0, N_OUTER)
    def _chunk(i):
        r0 = row_base + i * K
        o0 = out_base + i * CHUNK
        pltpu.sync_copy(idx_hbm.at[pl.ds(r0, K)], idx_v)
        copies = [
            pltpu.async_copy(
                table_hbm.at[idx_v.at[j]],
                rows_v.at[pl.ds(j * G, G)],
                sem,
            )
            for j in range(K)
        ]
        for c in copies:
            c.wait()
        pltpu.sync_copy(rows_v, out_hbm.at[pl.ds(o0, CHUNK)])


def kernel(token_ids, embedding_mat):
    idx2 = token_ids.reshape(-1, G).astype(jnp.int32)
    flat = _gather_kernel(idx2, embedding_mat)
    return flat.reshape(token_ids.shape + (DIM,))


# speedup vs baseline: 1.5012x; 1.0293x over previous
import functools

import jax
import jax.numpy as jnp
from jax import lax
from jax.experimental import pallas as pl
from jax.experimental.pallas import tpu as pltpu
from jax.experimental.pallas import tpu_sc as plsc

NUM_TOKENS = 4096 * 200
DIM = 32
NC = 2
NS = 16
NW = NC * NS
G = 128
K = 10
CHUNK = K * G
B_PER_W = NUM_TOKENS // NW
N_OUTER = B_PER_W // CHUNK
ROWS_PER_W = B_PER_W // G

_mesh = plsc.VectorSubcoreMesh(core_axis_name="c", subcore_axis_name="s")


@functools.partial(
    pl.kernel,
    out_type=jax.ShapeDtypeStruct((NUM_TOKENS, DIM), jnp.float32),
    mesh=_mesh,
    compiler_params=pltpu.CompilerParams(use_tc_tiling_on_sc=False),
    scratch_types=[
        pltpu.VMEM((ROWS_PER_W, G), jnp.int32),
        pltpu.VMEM((CHUNK, DIM), jnp.float32),
        pltpu.VMEM((CHUNK, DIM), jnp.float32),
        pltpu.SemaphoreType.DMA,
        pltpu.SemaphoreType.DMA,
        pltpu.SemaphoreType.DMA,
        pltpu.SemaphoreType.DMA,
    ],
)
def _gather_kernel(idx_hbm, table_hbm, out_hbm, idx_all, rows0, rows1,
                   sg0, sg1, ss0, ss1):
    wid = lax.axis_index("s") * NC + lax.axis_index("c")
    row_base = wid * ROWS_PER_W
    out_base = wid * B_PER_W

    pltpu.sync_copy(idx_hbm.at[pl.ds(row_base, ROWS_PER_W)], idx_all)

    bufs = ((rows0, sg0, ss0), (rows1, sg1, ss1))

    def fire_gathers(c, buf, sem):
        for j in range(K):
            pltpu.async_copy(
                table_hbm.at[idx_all.at[c * K + j]],
                buf.at[pl.ds(j * G, G)],
                sem,
            )

    def wait_gathers(buf, sem):
        pltpu.make_async_copy(out_hbm.at[pl.ds(0, CHUNK)], buf, sem).wait()

    def store(c, buf, sem):
        return pltpu.async_copy(
            buf, out_hbm.at[pl.ds(out_base + c * CHUNK, CHUNK)], sem)

    fire_gathers(0, rows0, sg0)
    fire_gathers(1, rows1, sg1)

    @pl.loop(0, N_OUTER - 2, step=2)
    def _pipe(i):
        for b in range(2):
            c = i + b
            buf, sg, ss = bufs[b]
            wait_gathers(buf, sg)
            store(c, buf, ss).wait()
            fire_gathers(c + 2, buf, sg)

    for b in range(2):
        c = N_OUTER - 2 + b
        buf, sg, ss = bufs[b]
        wait_gathers(buf, sg)
        store(c, buf, ss).wait()


def kernel(token_ids, embedding_mat):
    idx2 = token_ids.reshape(-1, G).astype(jnp.int32)
    flat = _gather_kernel(idx2, embedding_mat)
    return flat.reshape(token_ids.shape + (DIM,))


# speedup vs baseline: 1.5811x; 1.0532x over previous
import functools

import jax
import jax.numpy as jnp
from jax import lax
from jax.experimental import pallas as pl
from jax.experimental.pallas import tpu as pltpu
from jax.experimental.pallas import tpu_sc as plsc

NUM_TOKENS = 4096 * 200
DIM = 32
NC = 2
NS = 16
NW = NC * NS
G = 128
K = 10
CHUNK = K * G
B_PER_W = NUM_TOKENS // NW
N_OUTER = B_PER_W // CHUNK
ROWS_PER_W = B_PER_W // G

_mesh = plsc.VectorSubcoreMesh(core_axis_name="c", subcore_axis_name="s")


@functools.partial(
    pl.kernel,
    out_type=jax.ShapeDtypeStruct((NUM_TOKENS, DIM), jnp.float32),
    mesh=_mesh,
    compiler_params=pltpu.CompilerParams(use_tc_tiling_on_sc=False),
    scratch_types=[
        pltpu.VMEM((ROWS_PER_W, G), jnp.int32),
        pltpu.VMEM((CHUNK, DIM), jnp.float32),
        pltpu.VMEM((CHUNK, DIM), jnp.float32),
        pltpu.SemaphoreType.DMA,
        pltpu.SemaphoreType.DMA,
        pltpu.SemaphoreType.DMA,
        pltpu.SemaphoreType.DMA,
    ],
)
def _gather_kernel(idx_hbm, table_hbm, out_hbm, idx_all, rows0, rows1,
                   sg0, sg1, ss0, ss1):
    wid = lax.axis_index("s") * NC + lax.axis_index("c")
    row_base = wid * ROWS_PER_W
    out_base = wid * B_PER_W

    pltpu.sync_copy(idx_hbm.at[pl.ds(row_base, ROWS_PER_W)], idx_all)

    bufs = ((rows0, sg0, ss0), (rows1, sg1, ss1))

    def fire_gathers(c, buf, sem):
        for j in range(K):
            pltpu.async_copy(
                table_hbm.at[idx_all.at[c * K + j]],
                buf.at[pl.ds(j * G, G)],
                sem,
            )

    def wait_gathers(buf, sem):
        pltpu.make_async_copy(out_hbm.at[pl.ds(0, CHUNK)], buf, sem).wait()

    def store(c, buf, sem):
        return pltpu.async_copy(
            buf, out_hbm.at[pl.ds(out_base + c * CHUNK, CHUNK)], sem)

    fire_gathers(0, rows0, sg0)
    fire_gathers(1, rows1, sg1)

    @pl.loop(0, N_OUTER - 2, step=2)
    def _pipe(i):
        for b in range(2):
            c = i + b
            buf, sg, ss = bufs[b]
            wait_gathers(buf, sg)
            store(c, buf, ss).wait()
            fire_gathers(c + 2, buf, sg)

    for b in range(2):
        c = N_OUTER - 2 + b
        buf, sg, ss = bufs[b]
        wait_gathers(buf, sg)
        store(c, buf, ss).wait()


def kernel(token_ids, embedding_mat):
    nb, nt = token_ids.shape
    idx2 = token_ids.T.reshape(-1, G).astype(jnp.int32)
    flat = _gather_kernel(idx2, embedding_mat)
    return flat.reshape(nt, nb, DIM).transpose(1, 0, 2)
